# Initial kernel scaffold; baseline (speedup 1.0000x reference)
import jax
import jax.numpy as jnp
from jax.experimental import pallas as pl


def kernel(coords, features):
    raise NotImplementedError("write your pallas kernel here")


# speedup vs baseline: 2.2322x; 2.2322x over previous
import functools

import jax
import jax.numpy as jnp
from jax import lax
from jax.experimental import pallas as pl
from jax.experimental.pallas import tpu as pltpu
from jax.experimental.pallas import tpu_sc as plsc

N = 500000
TBL = 1 << 20
D = 16
P0, P1, P2 = 73856093, 19349663, 83492791

NPAD = 512000
ROWS_PER_TILE = NPAD // 16
HCHUNK = 2000
SCHUNK = 16000
QCHUNK = 128
NQ = (N + QCHUNK - 1) // QCHUNK
LAST_BASE = N - QCHUNK
NTILES = 32
P3_ITERS = (NQ + NTILES - 1) // NTILES
SLOTS = TBL // 16


def _fori(n, body):
    def b(i, carry):
        body(i, carry)
        return carry

    lax.fori_loop(jnp.int32(0), jnp.int32(n), b, jnp.int32(0))


def _body(coords_hbm, feats_hbm, out_hbm, table_hbm,
          cbuf, hbuf, tblv, sbuf, hq, wq, rq, rows, h_sp, sem):
    c = lax.axis_index("c")
    s = lax.axis_index("s")
    lane = lax.iota(jnp.int32, 16)

    def p1_chunk(k, _):
        row0 = s * ROWS_PER_TILE + k * HCHUNK

        def p1_vreg(j, _):
            r = (j * 16 + lane) * 3
            x = plsc.load_gather(cbuf, [r])
            y = plsc.load_gather(cbuf, [r + 1])
            z = plsc.load_gather(cbuf, [r + 2])
            h = (x * P0 + y * P1 + z * P2) & (TBL - 1)
            hbuf[pl.ds(j * 16, 16)] = h
            return 0

        pltpu.sync_copy(coords_hbm.at[pl.ds(row0 * 3, HCHUNK * 3)], cbuf)
        _fori(HCHUNK // 16, p1_vreg)
        pltpu.sync_copy(hbuf, h_sp.at[pl.ds(row0, HCHUNK)])
        return 0

    _fori(ROWS_PER_TILE // HCHUNK, p1_chunk)

    zero16 = jnp.zeros((16,), jnp.int32)

    def p2_zero(j, _):
        tblv[pl.ds(j * 16, 16)] = zero16
        return 0

    _fori(SLOTS // 16, p2_zero)
    plsc.subcore_barrier()

    def p2_chunk(kc, _):
        base = kc * SCHUNK
        pltpu.sync_copy(h_sp.at[pl.ds(base, SCHUNK)], sbuf)

        def p2_vreg(j, _):
            hv = sbuf[pl.ds(j * 16, 16)]
            iv = base + j * 16 + lane
            m = ((hv >> 16) == s) & (iv < N)
            plsc.store_scatter(tblv, [hv & (SLOTS - 1)], iv + 1, mask=m)
            return 0

        _fori(SCHUNK // 16, p2_vreg)
        return 0

    _fori(NPAD // SCHUNK, p2_chunk)
    pltpu.sync_copy(tblv, table_hbm.at[pl.ds(s * SLOTS, SLOTS)])
    plsc.subcore_barrier()

    wid = s * 2 + c

    def p3_chunk(j, _):
        q = wid + NTILES * j

        @pl.when(q < NQ)
        def _():
            base = jnp.minimum(q * QCHUNK, LAST_BASE)
            pltpu.sync_copy(h_sp.at[pl.ds(base, QCHUNK)], hq)
            pltpu.async_copy(table_hbm.at[hq], wq, sem).wait()

            def p3_fix(t, _):
                wv = wq[pl.ds(t * 16, 16)]
                rq[pl.ds(t * 16, 16)] = jnp.maximum(wv - 1, 0)
                return 0

            _fori(QCHUNK // 16, p3_fix)
            pltpu.async_copy(feats_hbm.at[rq], rows, sem).wait()
            pltpu.sync_copy(rows, out_hbm.at[pl.ds(base, QCHUNK)])

        return 0

    _fori(P3_ITERS, p3_chunk)


_sc_call = functools.partial(
    pl.kernel,
    out_type=[
        jax.ShapeDtypeStruct((N, D), jnp.float32),
        jax.ShapeDtypeStruct((TBL,), jnp.int32),
    ],
    mesh=plsc.VectorSubcoreMesh(core_axis_name="c", subcore_axis_name="s"),
    compiler_params=pltpu.CompilerParams(
        needs_layout_passes=False, use_tc_tiling_on_sc=False),
    scratch_types=[
        pltpu.VMEM((HCHUNK * 3,), jnp.int32),
        pltpu.VMEM((HCHUNK,), jnp.int32),
        pltpu.VMEM((SLOTS,), jnp.int32),
        pltpu.VMEM((SCHUNK,), jnp.int32),
        pltpu.VMEM((QCHUNK,), jnp.int32),
        pltpu.VMEM((QCHUNK,), jnp.int32),
        pltpu.VMEM((QCHUNK,), jnp.int32),
        pltpu.VMEM((QCHUNK, D), jnp.float32),
        pltpu.VMEM_SHARED((NPAD,), jnp.int32),
        pltpu.SemaphoreType.DMA,
    ],
)(_body)


def kernel(coords, features):
    coords32 = coords.astype(jnp.int32)
    coords_pad = jnp.pad(coords32, ((0, NPAD - N), (0, 0))).reshape(NPAD * 3)
    out, _ = _sc_call(coords_pad, features.astype(jnp.float32))
    return out
